# baseline (speedup 1.0000x reference)
import functools

import jax
import jax.numpy as jnp
from jax import lax
from jax.experimental import pallas as pl
from jax.experimental.pallas import tpu as pltpu
from jax.experimental.pallas import tpu_sc as plsc

N_FACTORS = 16
BATCH = 16384
NC = 2
NS = 16
NW = NC * NS
BPW = BATCH // NW
CHUNK = 128
LANES = 16


def _mf_body(iu_hbm, ii_hbm, utab_hbm, itab_hbm, w_hbm, b_hbm, out_hbm,
             iu_v, ii_v, u_rows, v_rows, w_v, b_v, out_v, sem):
    wid = lax.axis_index("s") * NC + lax.axis_index("c")
    base = wid * BPW

    pltpu.sync_copy(iu_hbm.at[pl.ds(base, BPW)], iu_v)
    pltpu.sync_copy(ii_hbm.at[pl.ds(base, BPW)], ii_v)
    pltpu.sync_copy(w_hbm, w_v)
    pltpu.sync_copy(b_hbm, b_v)
    wvec = w_v[...]
    b_b = b_v[...]

    copies = []
    for j in range(BPW // CHUNK):
        sl = pl.ds(j * CHUNK, CHUNK)
        copies.append(pltpu.async_copy(utab_hbm.at[iu_v.at[sl]], u_rows.at[sl], sem))
        copies.append(pltpu.async_copy(itab_hbm.at[ii_v.at[sl]], v_rows.at[sl], sem))
    for cp in copies:
        cp.wait()

    lanes = lax.iota(jnp.int32, LANES)
    w_b = [jnp.take_along_axis(wvec, jnp.full((LANES,), f, jnp.int32), axis=0)
           for f in range(N_FACTORS)]

    def block(g, carry):
        ridx = lanes + g * LANES
        acc = b_b
        for f in range(N_FACTORS):
            fidx = jnp.full((LANES,), f, jnp.int32)
            ucol = plsc.load_gather(u_rows, [ridx, fidx])
            vcol = plsc.load_gather(v_rows, [ridx, fidx])
            acc = acc + ucol * vcol * w_b[f]
        out_v[pl.ds(g * LANES, LANES)] = acc
        return carry

    lax.fori_loop(0, BPW // LANES, block, 0, unroll=2)

    pltpu.sync_copy(out_v, out_hbm.at[pl.ds(base, BPW)])


@jax.jit
def _mf_call(idx_users, idx_items, user_emb, item_emb, w, b):
    mesh = plsc.VectorSubcoreMesh(core_axis_name="c", subcore_axis_name="s")
    fn = pl.kernel(
        _mf_body,
        out_type=jax.ShapeDtypeStruct((BATCH,), jnp.float32),
        mesh=mesh,
        compiler_params=pltpu.CompilerParams(needs_layout_passes=False,
                                             use_tc_tiling_on_sc=False),
        scratch_types=[
            pltpu.VMEM((BPW,), jnp.int32),
            pltpu.VMEM((BPW,), jnp.int32),
            pltpu.VMEM((BPW, N_FACTORS), jnp.float32),
            pltpu.VMEM((BPW, N_FACTORS), jnp.float32),
            pltpu.VMEM((N_FACTORS,), jnp.float32),
            pltpu.VMEM((LANES,), jnp.float32),
            pltpu.VMEM((BPW,), jnp.float32),
            pltpu.SemaphoreType.DMA,
        ],
    )
    return fn(idx_users, idx_items, user_emb, item_emb, w, b)


def kernel(idx_users, idx_items, user_emb_mf, item_emb_mf, W_out, b_out):
    w_row = W_out.reshape((N_FACTORS,))
    b16 = jnp.broadcast_to(b_out.reshape(()), (LANES,))
    return _mf_call(idx_users.astype(jnp.int32), idx_items.astype(jnp.int32),
                    user_emb_mf, item_emb_mf, w_row, b16)


# speedup vs baseline: 1.4694x; 1.4694x over previous
import jax
import jax.numpy as jnp
from jax import lax
from jax.experimental import pallas as pl
from jax.experimental.pallas import tpu as pltpu
from jax.experimental.pallas import tpu_sc as plsc

N_FACTORS = 16
BATCH = 16384
NC = 2
NS = 16
NW = NC * NS
BPW = BATCH // NW
CH = 128
NCHUNK = BPW // CH
LANES = 16


def _mf_body(iu_hbm, ii_hbm, utab_hbm, itab_hbm, w_hbm, b_hbm, out_hbm,
             iu_v, ii_v, ub0, vb0, ub1, vb1, w_v, b_v, out_v, sem0, sem1):
    wid = lax.axis_index("s") * NC + lax.axis_index("c")
    base = wid * BPW

    pltpu.sync_copy(iu_hbm.at[pl.ds(base, BPW)], iu_v)
    pltpu.sync_copy(ii_hbm.at[pl.ds(base, BPW)], ii_v)
    pltpu.sync_copy(w_hbm, w_v)
    pltpu.sync_copy(b_hbm, b_v)
    wvec = w_v[...]
    b_b = b_v[...]

    lanes = lax.iota(jnp.int32, LANES)
    w_b = [jnp.take_along_axis(wvec, jnp.full((LANES,), f, jnp.int32), axis=0)
           for f in range(N_FACTORS)]

    def fire_chunk(k, ubuf, vbuf, sem):
        def fire(g, c):
            iu = iu_v[pl.ds(k * CH + g * LANES, LANES)]
            ii = ii_v[pl.ds(k * CH + g * LANES, LANES)]
            for l in range(LANES):
                j = g * LANES + l
                pltpu.async_copy(utab_hbm.at[iu[l]], ubuf.at[j], sem)
                pltpu.async_copy(itab_hbm.at[ii[l]], vbuf.at[j], sem)
            return c
        lax.fori_loop(0, CH // LANES, fire, 0)

    def drain(ubuf, vbuf, sem):
        pltpu.make_async_copy(utab_hbm.at[pl.ds(0, CH)], ubuf, sem).wait()
        pltpu.make_async_copy(itab_hbm.at[pl.ds(0, CH)], vbuf, sem).wait()

    def compute_chunk(k, ubuf, vbuf):
        def block(g, c):
            ridx = lanes + g * LANES
            acc = b_b
            for f in range(N_FACTORS):
                fidx = jnp.full((LANES,), f, jnp.int32)
                ucol = plsc.load_gather(ubuf, [ridx, fidx])
                vcol = plsc.load_gather(vbuf, [ridx, fidx])
                acc = acc + ucol * vcol * w_b[f]
            out_v[pl.ds(k * CH + g * LANES, LANES)] = acc
            return c
        lax.fori_loop(0, CH // LANES, block, 0)

    bufs = [(ub0, vb0, sem0), (ub1, vb1, sem1)]
    fire_chunk(0, *bufs[0])
    fire_chunk(1, *bufs[1])
    for k in range(NCHUNK):
        u, v, s = bufs[k % 2]
        drain(u, v, s)
        compute_chunk(k, u, v)
        if k + 2 < NCHUNK:
            fire_chunk(k + 2, u, v, s)

    pltpu.sync_copy(out_v, out_hbm.at[pl.ds(base, BPW)])


@jax.jit
def _mf_call(idx_users, idx_items, user_emb, item_emb, w, b):
    mesh = plsc.VectorSubcoreMesh(core_axis_name="c", subcore_axis_name="s")
    fn = pl.kernel(
        _mf_body,
        out_type=jax.ShapeDtypeStruct((BATCH,), jnp.float32),
        mesh=mesh,
        compiler_params=pltpu.CompilerParams(needs_layout_passes=False),
        scratch_types=[
            pltpu.VMEM((BPW,), jnp.int32),
            pltpu.VMEM((BPW,), jnp.int32),
            pltpu.VMEM((CH, N_FACTORS), jnp.float32),
            pltpu.VMEM((CH, N_FACTORS), jnp.float32),
            pltpu.VMEM((CH, N_FACTORS), jnp.float32),
            pltpu.VMEM((CH, N_FACTORS), jnp.float32),
            pltpu.VMEM((N_FACTORS,), jnp.float32),
            pltpu.VMEM((LANES,), jnp.float32),
            pltpu.VMEM((BPW,), jnp.float32),
            pltpu.SemaphoreType.DMA,
            pltpu.SemaphoreType.DMA,
        ],
    )
    return fn(idx_users, idx_items, user_emb, item_emb, w, b)


def kernel(idx_users, idx_items, user_emb_mf, item_emb_mf, W_out, b_out):
    w_row = W_out.reshape((N_FACTORS,))
    b16 = jnp.broadcast_to(b_out.reshape(()), (LANES,))
    return _mf_call(idx_users.astype(jnp.int32), idx_items.astype(jnp.int32),
                    user_emb_mf, item_emb_mf, w_row, b16)


# speedup vs baseline: 6.1503x; 4.1855x over previous
import jax
import jax.numpy as jnp
from jax import lax
from jax.experimental import pallas as pl
from jax.experimental.pallas import tpu as pltpu
from jax.experimental.pallas import tpu_sc as plsc

N_FACTORS = 16
BATCH = 16384
NC = 2
NS = 16
NW = NC * NS
BPW = BATCH // NW
LANES = 16
NCHUNK = BPW // LANES


def _mf_body(iu_hbm, ii_hbm, utabT_hbm, itabT_hbm, w_hbm, b_hbm, out_hbm,
             iu_v, ii_v, ub0, vb0, w_v, b_v, out_v, sem0):
    wid = lax.axis_index("s") * NC + lax.axis_index("c")
    base = wid * BPW

    pltpu.sync_copy(iu_hbm.at[pl.ds(base, BPW)], iu_v)
    pltpu.sync_copy(ii_hbm.at[pl.ds(base, BPW)], ii_v)
    pltpu.sync_copy(w_hbm, w_v)
    pltpu.sync_copy(b_hbm, b_v)
    wvec = w_v[...]
    b_b = b_v[...]

    lanes = lax.iota(jnp.int32, LANES)
    w_b = [jnp.take_along_axis(wvec, jnp.full((LANES,), f, jnp.int32), axis=0)
           for f in range(N_FACTORS)]

    def fire_chunk(k, ubuf, vbuf, sem):
        iu = iu_v[pl.ds(k * LANES, LANES)]
        ii = ii_v[pl.ds(k * LANES, LANES)]
        cu = lax.shift_left(lax.shift_right_logical(iu, 7), 7)
        ci = lax.shift_left(lax.shift_right_logical(ii, 7), 7)
        for l in range(LANES):
            co_u = pl.multiple_of(cu[l], 128)
            co_i = pl.multiple_of(ci[l], 128)
            pltpu.async_copy(utabT_hbm.at[:, pl.ds(co_u, 128)], ubuf.at[l], sem)
            pltpu.async_copy(itabT_hbm.at[:, pl.ds(co_i, 128)], vbuf.at[l], sem)

    def drain(ubuf, vbuf, sem):
        du = pltpu.make_async_copy(utabT_hbm.at[:, pl.ds(0, 128)], ubuf.at[0], sem)
        dv = pltpu.make_async_copy(itabT_hbm.at[:, pl.ds(0, 128)], vbuf.at[0], sem)
        for _ in range(LANES):
            du.wait()
            dv.wait()

    def compute_chunk(k, ubuf, vbuf):
        sl = pl.ds(k * LANES, LANES)
        ul = lax.bitwise_and(iu_v[sl], 127)
        il = lax.bitwise_and(ii_v[sl], 127)
        acc = b_b
        for f in range(N_FACTORS):
            fidx = jnp.full((LANES,), f, jnp.int32)
            ucol = plsc.load_gather(ubuf, [lanes, fidx, ul])
            vcol = plsc.load_gather(vbuf, [lanes, fidx, il])
            acc = acc + ucol * vcol * w_b[f]
        out_v[sl] = acc

    def step(k, carry):
        fire_chunk(k, ub0, vb0, sem0)
        drain(ub0, vb0, sem0)
        compute_chunk(k, ub0, vb0)
        return carry

    lax.fori_loop(0, NCHUNK, step, 0)

    pltpu.sync_copy(out_v, out_hbm.at[pl.ds(base, BPW)])


@jax.jit
def _mf_call(idx_users, idx_items, user_embT, item_embT, w, b):
    mesh = plsc.VectorSubcoreMesh(core_axis_name="c", subcore_axis_name="s")
    fn = pl.kernel(
        _mf_body,
        out_type=jax.ShapeDtypeStruct((BATCH,), jnp.float32),
        mesh=mesh,
        compiler_params=pltpu.CompilerParams(needs_layout_passes=False),
        scratch_types=[
            pltpu.VMEM((BPW,), jnp.int32),
            pltpu.VMEM((BPW,), jnp.int32),
            pltpu.VMEM((LANES, N_FACTORS, 128), jnp.float32),
            pltpu.VMEM((LANES, N_FACTORS, 128), jnp.float32),
            pltpu.VMEM((N_FACTORS,), jnp.float32),
            pltpu.VMEM((LANES,), jnp.float32),
            pltpu.VMEM((BPW,), jnp.float32),
            pltpu.SemaphoreType.DMA,
        ],
    )
    return fn(idx_users, idx_items, user_embT, item_embT, w, b)


def kernel(idx_users, idx_items, user_emb_mf, item_emb_mf, W_out, b_out):
    w_row = W_out.reshape((N_FACTORS,))
    b16 = jnp.broadcast_to(b_out.reshape(()), (LANES,))
    return _mf_call(idx_users.astype(jnp.int32), idx_items.astype(jnp.int32),
                    user_emb_mf.T, item_emb_mf.T, w_row, b16)
